# baseline (speedup 1.0000x reference)
import functools

import jax
import jax.numpy as jnp
from jax import lax
from jax.experimental import pallas as pl
from jax.experimental.pallas import tpu as pltpu
from jax.experimental.pallas import tpu_sc as plsc

_N = 10000
_E = 320000
_DF = 128
_DE = 16
_DO = 128

_NCORE = 2
_NTILE = 16
_CHUNK = 128
_N_PAD = 10240
_ROWS_PER_TILE = _N_PAD // _NTILE
_E_PAD = 327680
_CHUNKS_PER_TILE = _E_PAD // (_NCORE * _NTILE * _CHUNK)
_DEG_CHUNKS = _E_PAD // (_NTILE * _CHUNK)
_DUMMY = _N


def _sc_mesh():
    return plsc.VectorSubcoreMesh(core_axis_name="c", subcore_axis_name="s")


def _make_deg_kernel():
    @functools.partial(
        pl.kernel,
        out_type=(
            jax.ShapeDtypeStruct((_N_PAD,), jnp.float32),
            jax.ShapeDtypeStruct((_N_PAD,), jnp.float32),
            jax.ShapeDtypeStruct((_NCORE, _N_PAD, _DO), jnp.float32),
        ),
        mesh=_sc_mesh(),
        scratch_types=[
            pltpu.VMEM((40, _CHUNK), jnp.int32),
            pltpu.VMEM((40, _CHUNK), jnp.int32),
            pltpu.VMEM((_CHUNK,), jnp.float32),
            pltpu.VMEM((_CHUNK, _DO), jnp.float32),
            pltpu.VMEM((_CHUNK, _DO), jnp.float32),
            pltpu.VMEM_SHARED((_N_PAD,), jnp.float32),
            pltpu.VMEM_SHARED((_N_PAD, _DO), jnp.float32),
            pltpu.SemaphoreType.DMA,
            pltpu.SemaphoreType.DMA,
            pltpu.SemaphoreType.DMA,
            pltpu.SemaphoreType.DMA,
            pltpu.SemaphoreType.DMA,
        ],
    )
    def deg_kernel(src_ref, dst_ref, y_ref, zeros1_ref, zeros128_ref, ones_ref,
                   deg_out_ref, deg_in_ref, ya_ref,
                   idx_v, idx2_v, ones_v, y0, y1, deg_sh, ya_sh,
                   s1, sy0, sy1, ss0, ss1):
        c = lax.axis_index("c")
        s = lax.axis_index("s")
        row0 = s * _ROWS_PER_TILE
        rows = pl.ds(row0, _ROWS_PER_TILE)
        pltpu.sync_copy(zeros1_ref, deg_sh.at[rows])
        pltpu.sync_copy(zeros128_ref, ya_sh.at[rows])
        pltpu.sync_copy(ones_ref, ones_v)
        chunk0 = (c * _NTILE + s) * _CHUNKS_PER_TILE
        plsc.subcore_barrier()

        def scatter_ones(eref):
            for seg in range(4):
                pltpu.sync_copy(
                    eref.at[pl.ds(s * _DEG_CHUNKS + seg * 40, 40)], idx_v)

                def body(it, carry):
                    for u in range(8):
                        pltpu.async_copy(ones_v,
                                         deg_sh.at[idx_v.at[8 * it + u]], s1,
                                         add=True)
                    for u in range(8):
                        pltpu.make_async_copy(
                            ones_v, deg_sh.at[idx_v.at[8 * it + u]], s1).wait()
                    return carry

                lax.fori_loop(0, 5, body, 0)

        @pl.when(c == 0)
        def _():
            scatter_ones(src_ref)

        @pl.when(c == 1)
        def _():
            scatter_ones(dst_ref)

        def yload(j, buf, sem):
            pltpu.async_copy(
                y_ref.at[pl.ds((chunk0 + j) * _CHUNK, _CHUNK)], buf, sem)

        def yload_wait(j, buf, sem):
            pltpu.make_async_copy(
                y_ref.at[pl.ds((chunk0 + j) * _CHUNK, _CHUNK)], buf, sem).wait()

        def yscatter(j, buf, sem):
            pltpu.async_copy(buf, ya_sh.at[idx2_v.at[j]], sem, add=True)

        def yscatter_wait(j, buf, sem):
            pltpu.make_async_copy(buf, ya_sh.at[idx2_v.at[j]], sem).wait()

        for seg in range(2):
            off = seg * 40
            pltpu.sync_copy(dst_ref.at[pl.ds(chunk0 + off, 40)], idx2_v)
            yload(off, y0, sy0)

            def ybody(it, carry):
                a = 2 * it
                b = a + 1

                @pl.when(it > 0)
                def _():
                    yscatter_wait(b - 2, y1, ss1)

                yload(off + b, y1, sy1)
                yload_wait(off + a, y0, sy0)
                yscatter(a, y0, ss0)
                yload_wait(off + b, y1, sy1)
                yscatter_wait(a, y0, ss0)

                @pl.when(it < 19)
                def _():
                    yload(off + a + 2, y0, sy0)
                yscatter(b, y1, ss1)
                return carry

            lax.fori_loop(0, 20, ybody, 0)
            yscatter_wait(39, y1, ss1)

        plsc.subcore_barrier()

        @pl.when(c == 0)
        def _():
            pltpu.sync_copy(deg_sh.at[rows], deg_out_ref.at[rows])
            pltpu.sync_copy(ya_sh.at[rows], ya_ref.at[0].at[rows])

        @pl.when(c == 1)
        def _():
            pltpu.sync_copy(deg_sh.at[rows], deg_in_ref.at[rows])
            pltpu.sync_copy(ya_sh.at[rows], ya_ref.at[1].at[rows])

    return deg_kernel


def _make_main_kernel():
    @functools.partial(
        pl.kernel,
        out_type=jax.ShapeDtypeStruct((_NCORE, _N_PAD, _DF), jnp.float32),
        mesh=_sc_mesh(),
        scratch_types=[
            pltpu.VMEM((_CHUNKS_PER_TILE // 2, _CHUNK), jnp.int32),
            pltpu.VMEM((_CHUNKS_PER_TILE // 2, _CHUNK), jnp.int32),
            pltpu.VMEM((_CHUNK, _DF), jnp.float32),
            pltpu.VMEM((_CHUNK, _DF), jnp.float32),
            pltpu.VMEM_SHARED((_N_PAD, _DF), jnp.float32),
            pltpu.SemaphoreType.DMA,
            pltpu.SemaphoreType.DMA,
            pltpu.SemaphoreType.DMA,
            pltpu.SemaphoreType.DMA,
        ],
    )
    def main_kernel(feat_ref, src_ref, dst_ref, zeros128_ref,
                    h1_ref,
                    idx_src_v, idx_dst_v, g0, g1, h1_sh, sg0, sg1, ss0, ss1):
        c = lax.axis_index("c")
        s = lax.axis_index("s")
        row0 = s * _ROWS_PER_TILE
        rows = pl.ds(row0, _ROWS_PER_TILE)
        pltpu.sync_copy(zeros128_ref, h1_sh.at[rows])
        chunk0 = (c * _NTILE + s) * _CHUNKS_PER_TILE
        plsc.subcore_barrier()

        def gather(j, buf, sem):
            pltpu.async_copy(feat_ref.at[idx_src_v.at[j]], buf, sem)

        def gather_wait(j, buf, sem):
            pltpu.make_async_copy(feat_ref.at[idx_src_v.at[j]], buf, sem).wait()

        def scatter(j, buf, sem):
            pltpu.async_copy(buf, h1_sh.at[idx_dst_v.at[j]], sem, add=True)

        def scatter_wait(j, buf, sem):
            pltpu.make_async_copy(buf, h1_sh.at[idx_dst_v.at[j]], sem).wait()

        half = _CHUNKS_PER_TILE // 2
        n_pairs = half // 2

        for h in range(2):
            pltpu.sync_copy(src_ref.at[pl.ds(chunk0 + h * half, half)],
                            idx_src_v)
            pltpu.sync_copy(dst_ref.at[pl.ds(chunk0 + h * half, half)],
                            idx_dst_v)
            gather(0, g0, sg0)

            def body(it, carry):
                a = 2 * it
                b = a + 1

                @pl.when(it > 0)
                def _():
                    scatter_wait(b - 2, g1, ss1)

                gather(b, g1, sg1)
                gather_wait(a, g0, sg0)
                scatter(a, g0, ss0)
                gather_wait(b, g1, sg1)
                scatter_wait(a, g0, ss0)

                @pl.when(it < n_pairs - 1)
                def _():
                    gather(a + 2, g0, sg0)
                scatter(b, g1, ss1)
                return carry

            lax.fori_loop(0, n_pairs, body, 0)
            scatter_wait(half - 1, g1, ss1)

        plsc.subcore_barrier()

        @pl.when(c == 0)
        def _():
            pltpu.sync_copy(h1_sh.at[rows], h1_ref.at[0].at[rows])

        @pl.when(c == 1)
        def _():
            pltpu.sync_copy(h1_sh.at[rows], h1_ref.at[1].at[rows])

    return main_kernel


_EROWS_BLK = 2560
_Y_REAL_BLOCKS = _E // _EROWS_BLK


def _y_body(ef_ref, w2_ref, y_ref):
    i = pl.program_id(0)

    @pl.when(i < _Y_REAL_BLOCKS)
    def _():
        y_ref[...] = jnp.dot(ef_ref[...], w2_ref[...],
                             preferred_element_type=jnp.float32)

    @pl.when(i >= _Y_REAL_BLOCKS)
    def _():
        y_ref[...] = jnp.zeros_like(y_ref)


def _y_kernel(ef, w2):
    grid = (_E_PAD // _EROWS_BLK,)
    return pl.pallas_call(
        _y_body,
        grid=grid,
        in_specs=[
            pl.BlockSpec((_EROWS_BLK, _DE),
                         lambda i: (jnp.minimum(i, _Y_REAL_BLOCKS - 1), 0)),
            pl.BlockSpec((_DE, _DO), lambda i: (0, 0)),
        ],
        out_specs=pl.BlockSpec((_EROWS_BLK, _DO), lambda i: (i, 0)),
        out_shape=jax.ShapeDtypeStruct((_E_PAD, _DO), jnp.float32),
    )(ef, w2)


_ROWS_BLK = 1024


def _feat_body(x_ref, deg_ref, f_ref):
    norm = lax.rsqrt(jnp.maximum(deg_ref[...], 1.0))
    f_ref[...] = x_ref[...] * norm


def _feat_kernel(x_pad, deg_out_col):
    grid = (_N_PAD // _ROWS_BLK,)
    return pl.pallas_call(
        _feat_body,
        grid=grid,
        in_specs=[
            pl.BlockSpec((_ROWS_BLK, _DF), lambda i: (i, 0)),
            pl.BlockSpec((_ROWS_BLK, 1), lambda i: (i, 0)),
        ],
        out_specs=pl.BlockSpec((_ROWS_BLK, _DF), lambda i: (i, 0)),
        out_shape=jax.ShapeDtypeStruct((_N_PAD, _DF), jnp.float32),
    )(x_pad, deg_out_col)


def _mm_body(h1a_ref, h1b_ref, ya_ref, yb_ref, w1_ref,
             deg_ref, b_ref, out_ref):
    h1 = h1a_ref[0] + h1b_ref[0]
    acc = jnp.dot(h1, w1_ref[...], preferred_element_type=jnp.float32)
    acc = acc + ya_ref[0] + yb_ref[0]
    norm = lax.rsqrt(jnp.maximum(deg_ref[...], 1.0))
    out_ref[...] = acc * norm + b_ref[...]


_OUT_BLK = 2000


def _mm_kernel(h1, ya, w1, deg_in_col, b_row):
    grid = (_N // _OUT_BLK,)
    return pl.pallas_call(
        _mm_body,
        grid=grid,
        in_specs=[
            pl.BlockSpec((1, _OUT_BLK, _DF), lambda i: (0, i, 0)),
            pl.BlockSpec((1, _OUT_BLK, _DF), lambda i: (1, i, 0)),
            pl.BlockSpec((1, _OUT_BLK, _DO), lambda i: (0, i, 0)),
            pl.BlockSpec((1, _OUT_BLK, _DO), lambda i: (1, i, 0)),
            pl.BlockSpec((_DF, _DO), lambda i: (0, 0)),
            pl.BlockSpec((_OUT_BLK, 1), lambda i: (i, 0)),
            pl.BlockSpec((1, _DO), lambda i: (0, 0)),
        ],
        out_specs=pl.BlockSpec((_OUT_BLK, _DO), lambda i: (i, 0)),
        out_shape=jax.ShapeDtypeStruct((_N, _DO), jnp.float32),
    )(h1, h1, ya, ya, w1, deg_in_col, b_row)


def kernel(x, edge_feat, edge_index, W, b):
    src = edge_index[0].astype(jnp.int32)
    dst = edge_index[1].astype(jnp.int32)
    pad_e = _E_PAD - _E
    dummy = _DUMMY + (jnp.arange(pad_e, dtype=jnp.int32) % (_N_PAD - _N))
    src_r = jnp.concatenate([src, dummy]).reshape(-1, _CHUNK)
    dst_r = jnp.concatenate([dst, dummy]).reshape(-1, _CHUNK)

    x_pad = jnp.pad(x.astype(jnp.float32), ((0, _N_PAD - _N), (0, 0)))

    zeros1 = jnp.zeros((_ROWS_PER_TILE,), jnp.float32)
    ones128 = jnp.ones((_CHUNK,), jnp.float32)
    zeros128 = jnp.zeros((_ROWS_PER_TILE, _DF), jnp.float32)

    W = W.astype(jnp.float32)
    y = _y_kernel(edge_feat.astype(jnp.float32), W[_DF:])

    deg_out, deg_in, ya = _make_deg_kernel()(
        src_r, dst_r, y, zeros1, zeros128, ones128)

    feat = _feat_kernel(x_pad, deg_out.reshape(_N_PAD, 1))

    h1 = _make_main_kernel()(feat, src_r, dst_r, zeros128)

    return _mm_kernel(h1, ya, W[:_DF],
                      deg_in.reshape(_N_PAD, 1), b.reshape(1, _DO))


# speedup vs baseline: 1.0885x; 1.0885x over previous
import functools

import jax
import jax.numpy as jnp
from jax import lax
from jax.experimental import pallas as pl
from jax.experimental.pallas import tpu as pltpu
from jax.experimental.pallas import tpu_sc as plsc

_N = 10000
_E = 320000
_DF = 128
_DE = 16
_DO = 128

_NCORE = 2
_NTILE = 16
_CHUNK = 128
_N_PAD = 10240
_ROWS_PER_TILE = _N_PAD // _NTILE
_E_PAD = 327680
_CHUNKS_PER_TILE = _E_PAD // (_NCORE * _NTILE * _CHUNK)
_DEG_CHUNKS = _E_PAD // (_NTILE * _CHUNK)
_DUMMY = _N


def _sc_mesh():
    return plsc.VectorSubcoreMesh(core_axis_name="c", subcore_axis_name="s")


def _make_deg_kernel():
    @functools.partial(
        pl.kernel,
        out_type=(
            jax.ShapeDtypeStruct((_N_PAD,), jnp.float32),
            jax.ShapeDtypeStruct((_N_PAD,), jnp.float32),
            jax.ShapeDtypeStruct((_NCORE, _N_PAD, _DO), jnp.float32),
        ),
        mesh=_sc_mesh(),
        scratch_types=[
            pltpu.VMEM((40, _CHUNK), jnp.int32),
            pltpu.VMEM((40, _CHUNK), jnp.int32),
            pltpu.VMEM((_CHUNK,), jnp.float32),
            pltpu.VMEM((_CHUNK, _DO), jnp.float32),
            pltpu.VMEM((_CHUNK, _DO), jnp.float32),
            pltpu.VMEM_SHARED((_N_PAD,), jnp.float32),
            pltpu.VMEM_SHARED((_N_PAD, _DO), jnp.float32),
            pltpu.SemaphoreType.DMA,
            pltpu.SemaphoreType.DMA,
            pltpu.SemaphoreType.DMA,
            pltpu.SemaphoreType.DMA,
            pltpu.SemaphoreType.DMA,
        ],
    )
    def deg_kernel(src_ref, dst_ref, y_ref, zeros1_ref, zeros128_ref, ones_ref,
                   deg_out_ref, deg_in_ref, ya_ref,
                   idx_v, idx2_v, ones_v, y0, y1, deg_sh, ya_sh,
                   s1, sy0, sy1, ss0, ss1):
        c = lax.axis_index("c")
        s = lax.axis_index("s")
        row0 = s * _ROWS_PER_TILE
        rows = pl.ds(row0, _ROWS_PER_TILE)
        pltpu.sync_copy(zeros1_ref, deg_sh.at[rows])
        pltpu.sync_copy(zeros128_ref, ya_sh.at[rows])
        pltpu.sync_copy(ones_ref, ones_v)
        chunk0 = (c * _NTILE + s) * _CHUNKS_PER_TILE
        plsc.subcore_barrier()

        def scatter_ones(eref):
            for seg in range(4):
                pltpu.sync_copy(
                    eref.at[pl.ds(s * _DEG_CHUNKS + seg * 40, 40)], idx_v)

                def body(it, carry):
                    for u in range(8):
                        pltpu.async_copy(ones_v,
                                         deg_sh.at[idx_v.at[8 * it + u]], s1,
                                         add=True)
                    for u in range(8):
                        pltpu.make_async_copy(
                            ones_v, deg_sh.at[idx_v.at[8 * it + u]], s1).wait()
                    return carry

                lax.fori_loop(0, 5, body, 0)

        @pl.when(c == 0)
        def _():
            scatter_ones(src_ref)

        @pl.when(c == 1)
        def _():
            scatter_ones(dst_ref)

        def yload(j, buf, sem):
            pltpu.async_copy(
                y_ref.at[pl.ds((chunk0 + j) * _CHUNK, _CHUNK)], buf, sem)

        def yload_wait(j, buf, sem):
            pltpu.make_async_copy(
                y_ref.at[pl.ds((chunk0 + j) * _CHUNK, _CHUNK)], buf, sem).wait()

        def yscatter(j, buf, sem):
            pltpu.async_copy(buf, ya_sh.at[idx2_v.at[j]], sem, add=True)

        def yscatter_wait(j, buf, sem):
            pltpu.make_async_copy(buf, ya_sh.at[idx2_v.at[j]], sem).wait()

        for seg in range(2):
            off = seg * 40
            pltpu.sync_copy(dst_ref.at[pl.ds(chunk0 + off, 40)], idx2_v)
            yload(off, y0, sy0)

            def ybody(it, carry):
                a = 2 * it
                b = a + 1

                @pl.when(it > 0)
                def _():
                    yscatter_wait(b - 2, y1, ss1)

                yload(off + b, y1, sy1)
                yload_wait(off + a, y0, sy0)
                yscatter(a, y0, ss0)
                yload_wait(off + b, y1, sy1)
                yscatter_wait(a, y0, ss0)

                @pl.when(it < 19)
                def _():
                    yload(off + a + 2, y0, sy0)
                yscatter(b, y1, ss1)
                return carry

            lax.fori_loop(0, 20, ybody, 0)
            yscatter_wait(39, y1, ss1)

        plsc.subcore_barrier()

        @pl.when(c == 0)
        def _():
            pltpu.sync_copy(deg_sh.at[rows], deg_out_ref.at[rows])
            pltpu.sync_copy(ya_sh.at[rows], ya_ref.at[0].at[rows])

        @pl.when(c == 1)
        def _():
            pltpu.sync_copy(deg_sh.at[rows], deg_in_ref.at[rows])
            pltpu.sync_copy(ya_sh.at[rows], ya_ref.at[1].at[rows])

    return deg_kernel


def _make_main_kernel():
    @functools.partial(
        pl.kernel,
        out_type=jax.ShapeDtypeStruct((_NCORE, _N_PAD, _DF), jnp.float32),
        mesh=_sc_mesh(),
        scratch_types=[
            pltpu.VMEM((_CHUNKS_PER_TILE // 2, _CHUNK), jnp.int32),
            pltpu.VMEM((_CHUNKS_PER_TILE // 2, _CHUNK), jnp.int32),
            pltpu.VMEM((_CHUNK, _DF), jnp.float32),
            pltpu.VMEM((_CHUNK, _DF), jnp.float32),
            pltpu.VMEM_SHARED((_N_PAD, _DF), jnp.float32),
            pltpu.SemaphoreType.DMA,
            pltpu.SemaphoreType.DMA,
            pltpu.SemaphoreType.DMA,
            pltpu.SemaphoreType.DMA,
        ],
    )
    def main_kernel(feat_ref, src_ref, dst_ref, zeros128_ref,
                    h1_ref,
                    idx_src_v, idx_dst_v, g0, g1, h1_sh, sg0, sg1, ss0, ss1):
        c = lax.axis_index("c")
        s = lax.axis_index("s")
        row0 = s * _ROWS_PER_TILE
        rows = pl.ds(row0, _ROWS_PER_TILE)
        pltpu.sync_copy(zeros128_ref, h1_sh.at[rows])
        chunk0 = (c * _NTILE + s) * _CHUNKS_PER_TILE
        plsc.subcore_barrier()

        def gather(j, buf, sem):
            pltpu.async_copy(feat_ref.at[idx_src_v.at[j]], buf, sem)

        def gather_wait(j, buf, sem):
            pltpu.make_async_copy(feat_ref.at[idx_src_v.at[j]], buf, sem).wait()

        def scatter(j, buf, sem):
            pltpu.async_copy(buf, h1_sh.at[idx_dst_v.at[j]], sem, add=True)

        def scatter_wait(j, buf, sem):
            pltpu.make_async_copy(buf, h1_sh.at[idx_dst_v.at[j]], sem).wait()

        half = _CHUNKS_PER_TILE // 2
        n_pairs = half // 2

        for h in range(2):
            pltpu.sync_copy(src_ref.at[pl.ds(chunk0 + h * half, half)],
                            idx_src_v)
            pltpu.sync_copy(dst_ref.at[pl.ds(chunk0 + h * half, half)],
                            idx_dst_v)
            gather(0, g0, sg0)

            def body(it, carry):
                a = 2 * it
                b = a + 1

                @pl.when(it > 0)
                def _():
                    scatter_wait(b - 2, g1, ss1)

                gather(b, g1, sg1)
                gather_wait(a, g0, sg0)
                scatter(a, g0, ss0)
                gather_wait(b, g1, sg1)
                scatter_wait(a, g0, ss0)

                @pl.when(it < n_pairs - 1)
                def _():
                    gather(a + 2, g0, sg0)
                scatter(b, g1, ss1)
                return carry

            lax.fori_loop(0, n_pairs, body, 0)
            scatter_wait(half - 1, g1, ss1)

        plsc.subcore_barrier()

        @pl.when(c == 0)
        def _():
            pltpu.sync_copy(h1_sh.at[rows], h1_ref.at[0].at[rows])

        @pl.when(c == 1)
        def _():
            pltpu.sync_copy(h1_sh.at[rows], h1_ref.at[1].at[rows])

    return main_kernel


_EROWS_BLK = 8192
_Y_FULL_BLOCKS = _E // _EROWS_BLK
_Y_TAIL = _E - _Y_FULL_BLOCKS * _EROWS_BLK


def _y_body(ef_ref, tail_ref, w2_ref, y_ref):
    i = pl.program_id(0)

    @pl.when(i < _Y_FULL_BLOCKS)
    def _():
        y_ref[...] = jnp.dot(ef_ref[...], w2_ref[...],
                             preferred_element_type=jnp.float32)

    @pl.when(i == _Y_FULL_BLOCKS)
    def _():
        y_ref[...] = jnp.zeros_like(y_ref)
        y_ref[0:_Y_TAIL, :] = jnp.dot(tail_ref[...], w2_ref[...],
                                      preferred_element_type=jnp.float32)


def _y_kernel(ef, ef_tail, w2):
    grid = (_E_PAD // _EROWS_BLK,)
    return pl.pallas_call(
        _y_body,
        grid=grid,
        in_specs=[
            pl.BlockSpec((_EROWS_BLK, _DE),
                         lambda i: (jnp.minimum(i, _Y_FULL_BLOCKS - 1), 0)),
            pl.BlockSpec((_Y_TAIL, _DE), lambda i: (0, 0)),
            pl.BlockSpec((_DE, _DO), lambda i: (0, 0)),
        ],
        out_specs=pl.BlockSpec((_EROWS_BLK, _DO), lambda i: (i, 0)),
        out_shape=jax.ShapeDtypeStruct((_E_PAD, _DO), jnp.float32),
    )(ef, ef_tail, w2)


_ROWS_BLK = 1024


def _feat_body(x_ref, deg_ref, f_ref):
    norm = lax.rsqrt(jnp.maximum(deg_ref[...], 1.0))
    f_ref[...] = x_ref[...] * norm


def _feat_kernel(x_pad, deg_out_col):
    grid = (_N_PAD // _ROWS_BLK,)
    return pl.pallas_call(
        _feat_body,
        grid=grid,
        in_specs=[
            pl.BlockSpec((_ROWS_BLK, _DF), lambda i: (i, 0)),
            pl.BlockSpec((_ROWS_BLK, 1), lambda i: (i, 0)),
        ],
        out_specs=pl.BlockSpec((_ROWS_BLK, _DF), lambda i: (i, 0)),
        out_shape=jax.ShapeDtypeStruct((_N_PAD, _DF), jnp.float32),
    )(x_pad, deg_out_col)


def _mm_body(h1a_ref, h1b_ref, ya_ref, yb_ref, w1_ref,
             deg_ref, b_ref, out_ref):
    h1 = h1a_ref[0] + h1b_ref[0]
    acc = jnp.dot(h1, w1_ref[...], preferred_element_type=jnp.float32)
    acc = acc + ya_ref[0] + yb_ref[0]
    norm = lax.rsqrt(jnp.maximum(deg_ref[...], 1.0))
    out_ref[...] = acc * norm + b_ref[...]


_OUT_BLK = 2000


def _mm_kernel(h1, ya, w1, deg_in_col, b_row):
    grid = (_N // _OUT_BLK,)
    return pl.pallas_call(
        _mm_body,
        grid=grid,
        in_specs=[
            pl.BlockSpec((1, _OUT_BLK, _DF), lambda i: (0, i, 0)),
            pl.BlockSpec((1, _OUT_BLK, _DF), lambda i: (1, i, 0)),
            pl.BlockSpec((1, _OUT_BLK, _DO), lambda i: (0, i, 0)),
            pl.BlockSpec((1, _OUT_BLK, _DO), lambda i: (1, i, 0)),
            pl.BlockSpec((_DF, _DO), lambda i: (0, 0)),
            pl.BlockSpec((_OUT_BLK, 1), lambda i: (i, 0)),
            pl.BlockSpec((1, _DO), lambda i: (0, 0)),
        ],
        out_specs=pl.BlockSpec((_OUT_BLK, _DO), lambda i: (i, 0)),
        out_shape=jax.ShapeDtypeStruct((_N, _DO), jnp.float32),
    )(h1, h1, ya, ya, w1, deg_in_col, b_row)


def kernel(x, edge_feat, edge_index, W, b):
    src = edge_index[0].astype(jnp.int32)
    dst = edge_index[1].astype(jnp.int32)
    pad_e = _E_PAD - _E
    dummy = _DUMMY + (jnp.arange(pad_e, dtype=jnp.int32) % (_N_PAD - _N))
    src_r = jnp.concatenate([src, dummy]).reshape(-1, _CHUNK)
    dst_r = jnp.concatenate([dst, dummy]).reshape(-1, _CHUNK)

    x_pad = jnp.pad(x.astype(jnp.float32), ((0, _N_PAD - _N), (0, 0)))

    zeros1 = jnp.zeros((_ROWS_PER_TILE,), jnp.float32)
    ones128 = jnp.ones((_CHUNK,), jnp.float32)
    zeros128 = jnp.zeros((_ROWS_PER_TILE, _DF), jnp.float32)

    W = W.astype(jnp.float32)
    ef32 = edge_feat.astype(jnp.float32)
    y = _y_kernel(ef32, ef32[_Y_FULL_BLOCKS * _EROWS_BLK:], W[_DF:])

    deg_out, deg_in, ya = _make_deg_kernel()(
        src_r, dst_r, y, zeros1, zeros128, ones128)

    feat = _feat_kernel(x_pad, deg_out.reshape(_N_PAD, 1))

    h1 = _make_main_kernel()(feat, src_r, dst_r, zeros128)

    return _mm_kernel(h1, ya, W[:_DF],
                      deg_in.reshape(_N_PAD, 1), b.reshape(1, _DO))
